# baseline (speedup 1.0000x reference)
import math

import jax
import jax.numpy as jnp
from jax.experimental import pallas as pl
from jax.experimental.pallas import tpu as pltpu

TIMESTEPS = 1000
MAX_OUTLIERS = 128
N = 256
B = 8
SPS = 8
N_GOOD = 128
D = 256
DM = 256


def _ratio_table():
    epsilon = 0.008
    steps = jnp.linspace(0.0, TIMESTEPS, TIMESTEPS + 1, dtype=jnp.float32)
    f_t = jnp.cos((steps / TIMESTEPS + epsilon) / (1.0 + epsilon) * math.pi * 0.5) ** 2
    betas = jnp.clip(1.0 - f_t[1:] / f_t[:TIMESTEPS], 0.0, 0.999)
    alphas_cumprod = jnp.cumprod(1.0 - betas)
    tab = jnp.sqrt(1.0 - alphas_cumprod)
    tab = jnp.concatenate([tab, jnp.zeros((24,), jnp.float32)])
    return tab.reshape(8, 128)


def _kern(t_sref, g_ref, b_ref, pn_ref, tab_ref,
          Win, Wt, Wq, Wk, Wv, Wo, Wh, Wd1, Wd2, wc,
          pl_ref, gt_ref, pt_ref, gtt_ref):
    W = dict(Win=Win[...], Wq=Wq[...] * (1.0 / 16.0), Wk=Wk[...], Wv=Wv[...],
             Wo=Wo[...], Wh=Wh[...], Wd1=Wd1[...], Wd2=Wd2[...], wc=wc[...])
    tab = tab_ref[...]

    r8 = jax.lax.broadcasted_iota(jnp.int32, (8, 128), 0)
    c8 = jax.lax.broadcasted_iota(jnp.int32, (8, 128), 1)
    idx8 = r8 * 128 + c8
    ts, outliers, datanums = [], [], []
    for s in range(SPS):
        t = t_sref[s]
        ratio = jnp.sum(jnp.where(idx8 == t, tab, 0.0))
        outlier = jnp.floor(MAX_OUTLIERS * ratio).astype(jnp.int32)
        ts.append(t)
        outliers.append(outlier)
        datanums.append(N_GOOD + outlier)

    lane = jax.lax.broadcasted_iota(jnp.int32, (SPS, 128), 1).astype(jnp.float32)
    tf_col = jnp.stack([t.astype(jnp.float32) for t in ts])[:, None]
    ang = tf_col * jnp.exp(-(math.log(10000.0) / 128.0) * lane)
    temb_all = jnp.dot(jnp.concatenate([jnp.sin(ang), jnp.cos(ang)], axis=1),
                       Wt[...])
    tq = jnp.dot(temb_all, W['Wq'])
    tk = jnp.dot(temb_all, W['Wk'])
    tv = jnp.dot(temb_all, W['Wv'])

    _hi = jax.lax.Precision.HIGHEST
    WinWq = jnp.dot(W['Win'], W['Wq'], precision=_hi)
    WinWk = jnp.dot(W['Win'], W['Wk'], precision=_hi)
    WinWv = jnp.dot(W['Win'], W['Wv'], precision=_hi)

    Xg = g_ref[...].reshape(SPS * N_GOOD, D)
    Xb = b_ref[...].reshape(SPS * (N - N_GOOD), D)
    iob = jax.lax.broadcasted_iota(jnp.int32, (SPS * (N - N_GOOD), 1), 0)
    thr = jnp.zeros((SPS * (N - N_GOOD), 1), jnp.int32)
    sid = iob // (N - N_GOOD)
    for s in range(SPS):
        thr = jnp.where(sid == s, outliers[s], thr)
    Xb = jnp.where((iob - sid * (N - N_GOOD)) < thr, Xb, 1.0)

    XgI = Xg.astype(jnp.int32).astype(jnp.float32)
    XbI = Xb.astype(jnp.int32).astype(jnp.float32)

    fg = jnp.dot(XgI, W['Win'])
    fb = jnp.dot(XbI, W['Win'])
    qg = jnp.dot(XgI, WinWq)
    qb = jnp.dot(XbI, WinWq)
    kg = jnp.dot(XgI, WinWk)
    kb = jnp.dot(XbI, WinWk)
    vg = jnp.dot(XgI, WinWv)
    vb = jnp.dot(XbI, WinWv)

    h2g = jax.nn.relu(jnp.dot(jax.nn.relu(jnp.dot(Xg, W['Wd1'])), W['Wd2']))
    h2b = jax.nn.relu(jnp.dot(jax.nn.relu(jnp.dot(Xb, W['Wd1'])), W['Wd2']))

    io_i = jax.lax.broadcasted_iota(jnp.int32, (N, N), 0)
    io_j = jax.lax.broadcasted_iota(jnp.int32, (N, N), 1)
    ch = jax.lax.broadcasted_iota(jnp.int32, (N, 128), 1)
    icol = io_i[:, 0:1]
    io_jf = io_j.astype(jnp.float32)

    for s in range(SPS):
        sl = slice(s * N_GOOD, (s + 1) * N_GOOD)
        datanum = datanums[s]

        hmax = jnp.maximum(jnp.max(h2g[sl], axis=0, keepdims=True),
                           jnp.max(h2b[sl], axis=0, keepdims=True))
        hmean = (jnp.sum(h2g[sl], axis=0, keepdims=True) +
                 jnp.sum(h2b[sl], axis=0, keepdims=True)) * (1.0 / N)
        pooled = jnp.concatenate([hmax, hmean], axis=1)
        pt_row = jax.nn.sigmoid(jnp.dot(pooled, W['wc']))

        feats = jnp.concatenate([fg[sl], fb[sl]], axis=0) + temb_all[s:s + 1]
        q = jnp.concatenate([qg[sl], qb[sl]], axis=0) + tq[s:s + 1]
        k = jnp.concatenate([kg[sl], kb[sl]], axis=0) + tk[s:s + 1]
        v = jnp.concatenate([vg[sl], vb[sl]], axis=0) + tv[s:s + 1]
        scores = jax.lax.dot_general(q, k, (((1,), (1,)), ((), ())))
        e = jnp.exp(scores)
        den = jnp.sum(e, axis=1, keepdims=True)
        av = jnp.dot(e, v)
        out = feats + jnp.dot(av, W['Wo']) / den
        logits = jnp.dot(jax.nn.relu(out), W['Wh'])

        keys_j = (jnp.broadcast_to(pn_ref[s], (N, N)) +
                  jnp.where(io_j >= datanum, 1e6, 0.0))
        keys_i = keys_j.T
        before = (keys_j < keys_i) | ((keys_j == keys_i) & (io_j < io_i))
        rank = jnp.sum(before.astype(jnp.float32), axis=1, keepdims=True)
        Q = (rank == io_jf).astype(jnp.float32)

        lab = jnp.where(icol < N_GOOD, 0.0, jnp.where(icol < datanum, 1.0, -1.0))
        M = logits + jnp.where(ch == 2, lab, 0.0)

        final = jax.lax.dot_general(M, Q, (((0,), (0,)), ((), ())))

        pl_ref[s] = final[0:2, :]
        gt_ref[s] = jnp.round(final[2:3, :]).astype(jnp.int32)
        pt_ref[s] = pt_row[0, 0]
        gtt_ref[s] = ts[s].astype(jnp.float32) * (1.0 / TIMESTEPS)


def kernel(good_tokens, bad_tokens, t, perm_noise,
           W_in, W_t, W_q, W_k, W_v, W_o, W_head, W_d1, W_d2, w_cls):
    tab = _ratio_table()
    pn_row = perm_noise.reshape(B, 1, N)
    Wh128 = jnp.pad(W_head, ((0, 0), (0, 126)))
    wc128 = jnp.pad(w_cls, ((0, 0), (0, 127)))

    full2d = lambda s: pl.BlockSpec(s, lambda i, *_: (0, 0))
    per_b = lambda s: pl.BlockSpec(s, lambda i, *_: (i, 0, 0))

    grid_spec = pltpu.PrefetchScalarGridSpec(
        num_scalar_prefetch=1,
        grid=(B // SPS,),
        in_specs=[
            per_b((SPS, N_GOOD, D)),
            per_b((SPS, N - N_GOOD, D)),
            per_b((SPS, 1, N)),
            full2d((8, 128)),
            full2d((D, DM)),
            full2d((DM, DM)),
            full2d((DM, DM)),
            full2d((DM, DM)),
            full2d((DM, DM)),
            full2d((DM, DM)),
            full2d((DM, 128)),
            full2d((D, 128)),
            full2d((128, 128)),
            full2d((256, 128)),
        ],
        out_specs=[
            per_b((SPS, 2, N)),
            per_b((SPS, 1, N)),
            pl.BlockSpec(memory_space=pltpu.SMEM),
            pl.BlockSpec(memory_space=pltpu.SMEM),
        ],
    )

    pred_label, gt3, pred_t, gt_t = pl.pallas_call(
        _kern,
        grid_spec=grid_spec,
        out_shape=[
            jax.ShapeDtypeStruct((B, 2, N), jnp.float32),
            jax.ShapeDtypeStruct((B, 1, N), jnp.int32),
            jax.ShapeDtypeStruct((B,), jnp.float32),
            jax.ShapeDtypeStruct((B,), jnp.float32),
        ],
    )(t, good_tokens, bad_tokens, pn_row, tab,
      W_in, W_t, W_q, W_k, W_v, W_o, Wh128, W_d1, W_d2, wc128)

    gt_label = gt3.reshape(B, N)
    return pred_label, gt_label, pred_t, gt_t


# speedup vs baseline: 1.0156x; 1.0156x over previous
import math

import jax
import jax.numpy as jnp
from jax.experimental import pallas as pl
from jax.experimental.pallas import tpu as pltpu

TIMESTEPS = 1000
MAX_OUTLIERS = 128
N = 256
B = 8
SPS = 8
N_GOOD = 128
D = 256
DM = 256


def _ratio_table():
    epsilon = 0.008
    steps = jnp.linspace(0.0, TIMESTEPS, TIMESTEPS + 1, dtype=jnp.float32)
    f_t = jnp.cos((steps / TIMESTEPS + epsilon) / (1.0 + epsilon) * math.pi * 0.5) ** 2
    betas = jnp.clip(1.0 - f_t[1:] / f_t[:TIMESTEPS], 0.0, 0.999)
    alphas_cumprod = jnp.cumprod(1.0 - betas)
    tab = jnp.sqrt(1.0 - alphas_cumprod)
    tab = jnp.concatenate([tab, jnp.zeros((24,), jnp.float32)])
    return tab.reshape(8, 128)


def _kern(t_sref, g_ref, b_ref, pn_ref, tab_ref,
          Win, Wt, Wq, Wk, Wv, Wo, Wh, Wd1, Wd2, wc,
          pl_ref, gt_ref, pt_ref, gtt_ref):
    W = dict(Win=Win[...], Wq=Wq[...] * (1.0 / 16.0), Wk=Wk[...], Wv=Wv[...],
             Wo=Wo[...], Wh=Wh[...], Wd1=Wd1[...], Wd2=Wd2[...], wc=wc[...])
    tab = tab_ref[...]

    r8 = jax.lax.broadcasted_iota(jnp.int32, (8, 128), 0)
    c8 = jax.lax.broadcasted_iota(jnp.int32, (8, 128), 1)
    idx8 = r8 * 128 + c8
    ts, outliers, datanums = [], [], []
    for s in range(SPS):
        t = t_sref[s]
        ratio = jnp.sum(jnp.where(idx8 == t, tab, 0.0))
        outlier = jnp.floor(MAX_OUTLIERS * ratio).astype(jnp.int32)
        ts.append(t)
        outliers.append(outlier)
        datanums.append(N_GOOD + outlier)

    lane = jax.lax.broadcasted_iota(jnp.int32, (SPS, 128), 1).astype(jnp.float32)
    tf_col = jnp.stack([t.astype(jnp.float32) for t in ts])[:, None]
    ang = tf_col * jnp.exp(-(math.log(10000.0) / 128.0) * lane)
    temb_all = jnp.dot(jnp.concatenate([jnp.sin(ang), jnp.cos(ang)], axis=1),
                       Wt[...])
    tq = jnp.dot(temb_all, W['Wq'])
    tk = jnp.dot(temb_all, W['Wk'])
    tv = jnp.dot(temb_all, W['Wv'])

    WinWq = jnp.dot(W['Win'], W['Wq'])
    WinWk = jnp.dot(W['Win'], W['Wk'])
    WinWv = jnp.dot(W['Win'], W['Wv'])

    Xg = g_ref[...].reshape(SPS * N_GOOD, D)
    Xb = b_ref[...].reshape(SPS * (N - N_GOOD), D)
    iob = jax.lax.broadcasted_iota(jnp.int32, (SPS * (N - N_GOOD), 1), 0)
    thr = jnp.zeros((SPS * (N - N_GOOD), 1), jnp.int32)
    sid = iob // (N - N_GOOD)
    for s in range(SPS):
        thr = jnp.where(sid == s, outliers[s], thr)
    Xb = jnp.where((iob - sid * (N - N_GOOD)) < thr, Xb, 1.0)

    XgI = Xg.astype(jnp.int32).astype(jnp.float32)
    XbI = Xb.astype(jnp.int32).astype(jnp.float32)

    fg = jnp.dot(XgI, W['Win'])
    fb = jnp.dot(XbI, W['Win'])
    qg = jnp.dot(XgI, WinWq)
    qb = jnp.dot(XbI, WinWq)
    kg = jnp.dot(XgI, WinWk)
    kb = jnp.dot(XbI, WinWk)
    vg = jnp.dot(XgI, WinWv)
    vb = jnp.dot(XbI, WinWv)

    h2g = jax.nn.relu(jnp.dot(jax.nn.relu(jnp.dot(Xg, W['Wd1'])), W['Wd2']))
    h2b = jax.nn.relu(jnp.dot(jax.nn.relu(jnp.dot(Xb, W['Wd1'])), W['Wd2']))

    io_i = jax.lax.broadcasted_iota(jnp.int32, (N, N), 0)
    io_j = jax.lax.broadcasted_iota(jnp.int32, (N, N), 1)
    ch = jax.lax.broadcasted_iota(jnp.int32, (N, 128), 1)
    icol = io_i[:, 0:1]
    io_jf = io_j.astype(jnp.float32)

    for s in range(SPS):
        sl = slice(s * N_GOOD, (s + 1) * N_GOOD)
        datanum = datanums[s]

        hmax = jnp.maximum(jnp.max(h2g[sl], axis=0, keepdims=True),
                           jnp.max(h2b[sl], axis=0, keepdims=True))
        hmean = (jnp.sum(h2g[sl], axis=0, keepdims=True) +
                 jnp.sum(h2b[sl], axis=0, keepdims=True)) * (1.0 / N)
        pooled = jnp.concatenate([hmax, hmean], axis=1)
        pt_row = jax.nn.sigmoid(jnp.dot(pooled, W['wc']))

        feats = jnp.concatenate([fg[sl], fb[sl]], axis=0) + temb_all[s:s + 1]
        q = jnp.concatenate([qg[sl], qb[sl]], axis=0) + tq[s:s + 1]
        k = jnp.concatenate([kg[sl], kb[sl]], axis=0) + tk[s:s + 1]
        v = jnp.concatenate([vg[sl], vb[sl]], axis=0) + tv[s:s + 1]
        scores = jax.lax.dot_general(q, k, (((1,), (1,)), ((), ())))
        e = jnp.exp(scores)
        den = jnp.sum(e, axis=1, keepdims=True)
        av = jnp.dot(e, v)
        out = feats + jnp.dot(av, W['Wo']) / den
        logits = jnp.dot(jax.nn.relu(out), W['Wh'])

        keys_j = (jnp.broadcast_to(pn_ref[s], (N, N)) +
                  jnp.where(io_j >= datanum, 1e6, 0.0))
        keys_i = keys_j.T
        before = (keys_j < keys_i) | ((keys_j == keys_i) & (io_j < io_i))
        rank = jnp.sum(before.astype(jnp.float32), axis=1, keepdims=True)
        Q = (rank == io_jf).astype(jnp.float32)

        lab = jnp.where(icol < N_GOOD, 0.0, jnp.where(icol < datanum, 1.0, -1.0))
        M = logits + jnp.where(ch == 2, lab, 0.0)

        final = jax.lax.dot_general(M, Q, (((0,), (0,)), ((), ())))

        pl_ref[s] = final[0:2, :]
        gt_ref[s] = jnp.round(final[2:3, :]).astype(jnp.int32)
        pt_ref[s] = pt_row[0, 0]
        gtt_ref[s] = ts[s].astype(jnp.float32) * (1.0 / TIMESTEPS)


def kernel(good_tokens, bad_tokens, t, perm_noise,
           W_in, W_t, W_q, W_k, W_v, W_o, W_head, W_d1, W_d2, w_cls):
    tab = _ratio_table()
    pn_row = perm_noise.reshape(B, 1, N)
    Wh128 = jnp.pad(W_head, ((0, 0), (0, 126)))
    wc128 = jnp.pad(w_cls, ((0, 0), (0, 127)))

    full2d = lambda s: pl.BlockSpec(s, lambda i, *_: (0, 0))
    per_b = lambda s: pl.BlockSpec(s, lambda i, *_: (i, 0, 0))

    grid_spec = pltpu.PrefetchScalarGridSpec(
        num_scalar_prefetch=1,
        grid=(B // SPS,),
        in_specs=[
            per_b((SPS, N_GOOD, D)),
            per_b((SPS, N - N_GOOD, D)),
            per_b((SPS, 1, N)),
            full2d((8, 128)),
            full2d((D, DM)),
            full2d((DM, DM)),
            full2d((DM, DM)),
            full2d((DM, DM)),
            full2d((DM, DM)),
            full2d((DM, DM)),
            full2d((DM, 128)),
            full2d((D, 128)),
            full2d((128, 128)),
            full2d((256, 128)),
        ],
        out_specs=[
            per_b((SPS, 2, N)),
            per_b((SPS, 1, N)),
            pl.BlockSpec(memory_space=pltpu.SMEM),
            pl.BlockSpec(memory_space=pltpu.SMEM),
        ],
    )

    pred_label, gt3, pred_t, gt_t = pl.pallas_call(
        _kern,
        grid_spec=grid_spec,
        out_shape=[
            jax.ShapeDtypeStruct((B, 2, N), jnp.float32),
            jax.ShapeDtypeStruct((B, 1, N), jnp.int32),
            jax.ShapeDtypeStruct((B,), jnp.float32),
            jax.ShapeDtypeStruct((B,), jnp.float32),
        ],
    )(t, good_tokens, bad_tokens, pn_row, tab,
      W_in, W_t, W_q, W_k, W_v, W_o, Wh128, W_d1, W_d2, wc128)

    gt_label = gt3.reshape(B, N)
    return pred_label, gt_label, pred_t, gt_t


# speedup vs baseline: 1.0700x; 1.0535x over previous
import math

import jax
import jax.numpy as jnp
from jax.experimental import pallas as pl
from jax.experimental.pallas import tpu as pltpu

TIMESTEPS = 1000
MAX_OUTLIERS = 128
N = 256
B = 8
SPS = 8
N_GOOD = 128
D = 256
DM = 256


def _ratio_table():
    epsilon = 0.008
    steps = jnp.linspace(0.0, TIMESTEPS, TIMESTEPS + 1, dtype=jnp.float32)
    f_t = jnp.cos((steps / TIMESTEPS + epsilon) / (1.0 + epsilon) * math.pi * 0.5) ** 2
    betas = jnp.clip(1.0 - f_t[1:] / f_t[:TIMESTEPS], 0.0, 0.999)
    alphas_cumprod = jnp.cumprod(1.0 - betas)
    tab = jnp.sqrt(1.0 - alphas_cumprod)
    tab = jnp.concatenate([tab, jnp.zeros((24,), jnp.float32)])
    return tab.reshape(8, 128)


def _kern(t_sref, g_ref, b_ref, pn_ref, tab_ref,
          Win, Wt, Wq, Wk, Wv, Wo, Wh, Wd1, Wd2, wc,
          pl_ref, gt_ref, pt_ref, gtt_ref):
    W = dict(Win=Win[...], Wq=Wq[...] * (1.0 / 16.0), Wk=Wk[...], Wv=Wv[...],
             Wo=Wo[...], Wh=Wh[...], Wd1=Wd1[...], Wd2=Wd2[...], wc=wc[...])
    tab = tab_ref[...]

    r8 = jax.lax.broadcasted_iota(jnp.int32, (8, 128), 0)
    c8 = jax.lax.broadcasted_iota(jnp.int32, (8, 128), 1)
    idx8 = r8 * 128 + c8
    ts, outliers, datanums = [], [], []
    for s in range(SPS):
        t = t_sref[s]
        ratio = jnp.sum(jnp.where(idx8 == t, tab, 0.0))
        outlier = jnp.floor(MAX_OUTLIERS * ratio).astype(jnp.int32)
        ts.append(t)
        outliers.append(outlier)
        datanums.append(N_GOOD + outlier)

    lane = jax.lax.broadcasted_iota(jnp.int32, (SPS, 128), 1).astype(jnp.float32)
    tf_col = jnp.stack([t.astype(jnp.float32) for t in ts])[:, None]
    ang = tf_col * jnp.exp(-(math.log(10000.0) / 128.0) * lane)
    temb_all = jnp.dot(jnp.concatenate([jnp.sin(ang), jnp.cos(ang)], axis=1),
                       Wt[...])
    tq = jnp.dot(temb_all, W['Wq'])
    tk = jnp.dot(temb_all, W['Wk'])
    tv = jnp.dot(temb_all, W['Wv'])

    WinWq = jnp.dot(W['Win'], W['Wq'])
    WinWk = jnp.dot(W['Win'], W['Wk'])
    WinWv = jnp.dot(W['Win'], W['Wv'])

    Xg = g_ref[...].reshape(SPS * N_GOOD, D)
    Xb = b_ref[...].reshape(SPS * (N - N_GOOD), D)
    iob = jax.lax.broadcasted_iota(jnp.int32, (SPS * (N - N_GOOD), 1), 0)
    thr = jnp.zeros((SPS * (N - N_GOOD), 1), jnp.int32)
    sid = iob // (N - N_GOOD)
    for s in range(SPS):
        thr = jnp.where(sid == s, outliers[s], thr)
    Xb = jnp.where((iob - sid * (N - N_GOOD)) < thr, Xb, 1.0)

    XgI = Xg.astype(jnp.int32).astype(jnp.float32)
    XbI = Xb.astype(jnp.int32).astype(jnp.float32)

    fg = jnp.dot(XgI, W['Win'])
    fb = jnp.dot(XbI, W['Win'])
    qg = jnp.dot(XgI, WinWq)
    qb = jnp.dot(XbI, WinWq)
    kg = jnp.dot(XgI, WinWk)
    kb = jnp.dot(XbI, WinWk)
    vg = jnp.dot(XgI, WinWv)
    vb = jnp.dot(XbI, WinWv)

    h2g = jax.nn.relu(jnp.dot(jax.nn.relu(jnp.dot(Xg, W['Wd1'])), W['Wd2']))
    h2b = jax.nn.relu(jnp.dot(jax.nn.relu(jnp.dot(Xb, W['Wd1'])), W['Wd2']))

    io_i = jax.lax.broadcasted_iota(jnp.int32, (N, N), 0)
    io_j = jax.lax.broadcasted_iota(jnp.int32, (N, N), 1)
    ch = jax.lax.broadcasted_iota(jnp.int32, (N, 128), 1)
    icol = io_i[:, 0:1]
    io_jf = io_j.astype(jnp.float32)

    for s in range(SPS):
        sl = slice(s * N_GOOD, (s + 1) * N_GOOD)
        datanum = datanums[s]

        hmax = jnp.maximum(jnp.max(h2g[sl], axis=0, keepdims=True),
                           jnp.max(h2b[sl], axis=0, keepdims=True))
        hmean = (jnp.sum(h2g[sl], axis=0, keepdims=True) +
                 jnp.sum(h2b[sl], axis=0, keepdims=True)) * (1.0 / N)
        pooled = jnp.concatenate([hmax, hmean], axis=1)
        pt_row = jax.nn.sigmoid(jnp.dot(pooled, W['wc']))

        feats = jnp.concatenate([fg[sl], fb[sl]], axis=0) + temb_all[s:s + 1]
        q = jnp.concatenate([qg[sl], qb[sl]], axis=0) + tq[s:s + 1]
        k = jnp.concatenate([kg[sl], kb[sl]], axis=0) + tk[s:s + 1]
        v = jnp.concatenate([vg[sl], vb[sl]], axis=0) + tv[s:s + 1]
        scores = jax.lax.dot_general(q, k, (((1,), (1,)), ((), ())))
        e = jnp.exp(scores)
        den = jnp.sum(e, axis=1, keepdims=True)
        av = jnp.dot(e, v)
        out = feats + jnp.dot(av, W['Wo']) / den
        logits = jnp.dot(jax.nn.relu(out), W['Wh'])

        keys_j = (jnp.broadcast_to(pn_ref[s], (N, N)) +
                  jnp.where(io_j >= datanum, 1e6, 0.0))
        keys_i = keys_j.T
        before = (keys_j < keys_i) | ((keys_j == keys_i) & (io_j < io_i))
        rank = jnp.sum(before.astype(jnp.float32), axis=1, keepdims=True)
        Q = (rank == io_jf).astype(jnp.float32)

        lab = jnp.where(icol < N_GOOD, 0.0, jnp.where(icol < datanum, 1.0, -1.0))
        M = logits[:, 0:8] + jnp.where(ch[:, 0:8] == 2, lab, 0.0)

        final = jax.lax.dot_general(M, Q, (((0,), (0,)), ((), ())))

        pl_ref[s] = final[0:2, :]
        gt_ref[s] = jnp.round(final[2:3, :]).astype(jnp.int32)
        pt_ref[s] = pt_row[0, 0]
        gtt_ref[s] = ts[s].astype(jnp.float32) * (1.0 / TIMESTEPS)


def kernel(good_tokens, bad_tokens, t, perm_noise,
           W_in, W_t, W_q, W_k, W_v, W_o, W_head, W_d1, W_d2, w_cls):
    tab = _ratio_table()
    pn_row = perm_noise.reshape(B, 1, N)
    Wh128 = jnp.pad(W_head, ((0, 0), (0, 126)))
    wc128 = jnp.pad(w_cls, ((0, 0), (0, 127)))

    full2d = lambda s: pl.BlockSpec(s, lambda i, *_: (0, 0))
    per_b = lambda s: pl.BlockSpec(s, lambda i, *_: (i, 0, 0))

    grid_spec = pltpu.PrefetchScalarGridSpec(
        num_scalar_prefetch=1,
        grid=(B // SPS,),
        in_specs=[
            per_b((SPS, N_GOOD, D)),
            per_b((SPS, N - N_GOOD, D)),
            per_b((SPS, 1, N)),
            full2d((8, 128)),
            full2d((D, DM)),
            full2d((DM, DM)),
            full2d((DM, DM)),
            full2d((DM, DM)),
            full2d((DM, DM)),
            full2d((DM, DM)),
            full2d((DM, 128)),
            full2d((D, 128)),
            full2d((128, 128)),
            full2d((256, 128)),
        ],
        out_specs=[
            per_b((SPS, 2, N)),
            per_b((SPS, 1, N)),
            pl.BlockSpec(memory_space=pltpu.SMEM),
            pl.BlockSpec(memory_space=pltpu.SMEM),
        ],
    )

    pred_label, gt3, pred_t, gt_t = pl.pallas_call(
        _kern,
        grid_spec=grid_spec,
        out_shape=[
            jax.ShapeDtypeStruct((B, 2, N), jnp.float32),
            jax.ShapeDtypeStruct((B, 1, N), jnp.int32),
            jax.ShapeDtypeStruct((B,), jnp.float32),
            jax.ShapeDtypeStruct((B,), jnp.float32),
        ],
    )(t, good_tokens, bad_tokens, pn_row, tab,
      W_in, W_t, W_q, W_k, W_v, W_o, Wh128, W_d1, W_d2, wc128)

    gt_label = gt3.reshape(B, N)
    return pred_label, gt_label, pred_t, gt_t


# speedup vs baseline: 1.0783x; 1.0078x over previous
import math

import jax
import jax.numpy as jnp
from jax.experimental import pallas as pl
from jax.experimental.pallas import tpu as pltpu

TIMESTEPS = 1000
MAX_OUTLIERS = 128
N = 256
B = 8
SPS = 8
N_GOOD = 128
D = 256
DM = 256


def _ratio_table():
    epsilon = 0.008
    steps = jnp.linspace(0.0, TIMESTEPS, TIMESTEPS + 1, dtype=jnp.float32)
    f_t = jnp.cos((steps / TIMESTEPS + epsilon) / (1.0 + epsilon) * math.pi * 0.5) ** 2
    betas = jnp.clip(1.0 - f_t[1:] / f_t[:TIMESTEPS], 0.0, 0.999)
    alphas_cumprod = jnp.cumprod(1.0 - betas)
    tab = jnp.sqrt(1.0 - alphas_cumprod)
    tab = jnp.concatenate([tab, jnp.zeros((24,), jnp.float32)])
    return tab.reshape(8, 128)


def _kern(t_sref, g_ref, b_ref, pn_ref, tab_ref,
          Win, Wt, Wq, Wk, Wv, Wo, Wh, Wd1, Wd2, wc,
          pl_ref, gt_ref, pt_ref, gtt_ref):
    W = dict(Win=Win[...], Wq=Wq[...] * (1.0 / 16.0), Wk=Wk[...], Wv=Wv[...],
             Wo=Wo[...], Wh=Wh[...], Wd1=Wd1[...], Wd2=Wd2[...], wc=wc[...])
    tab = tab_ref[...]

    r8 = jax.lax.broadcasted_iota(jnp.int32, (8, 128), 0)
    c8 = jax.lax.broadcasted_iota(jnp.int32, (8, 128), 1)
    idx8 = r8 * 128 + c8
    ts, outliers, datanums = [], [], []
    for s in range(SPS):
        t = t_sref[s]
        ratio = jnp.sum(jnp.where(idx8 == t, tab, 0.0))
        outlier = jnp.floor(MAX_OUTLIERS * ratio).astype(jnp.int32)
        ts.append(t)
        outliers.append(outlier)
        datanums.append(N_GOOD + outlier)

    lane = jax.lax.broadcasted_iota(jnp.int32, (SPS, 128), 1).astype(jnp.float32)
    tf_col = jnp.stack([t.astype(jnp.float32) for t in ts])[:, None]
    ang = tf_col * jnp.exp(-(math.log(10000.0) / 128.0) * lane)
    temb_all = jnp.dot(jnp.concatenate([jnp.sin(ang), jnp.cos(ang)], axis=1),
                       Wt[...])
    tq = jnp.dot(temb_all, W['Wq'])
    tk = jnp.dot(temb_all, W['Wk'])
    tv = jnp.dot(temb_all, W['Wv'])

    WinWq = jnp.dot(W['Win'], W['Wq'])
    WinWk = jnp.dot(W['Win'], W['Wk'])
    WinWv = jnp.dot(W['Win'], W['Wv'])

    Xg = g_ref[...].reshape(SPS * N_GOOD, D)
    Xb = b_ref[...].reshape(SPS * (N - N_GOOD), D)
    iob = jax.lax.broadcasted_iota(jnp.int32, (SPS * (N - N_GOOD), 1), 0)
    thr = jnp.zeros((SPS * (N - N_GOOD), 1), jnp.int32)
    sid = iob // (N - N_GOOD)
    for s in range(SPS):
        thr = jnp.where(sid == s, outliers[s], thr)
    Xb = jnp.where((iob - sid * (N - N_GOOD)) < thr, Xb, 1.0)

    XgI = Xg.astype(jnp.int32).astype(jnp.float32)
    XbI = Xb.astype(jnp.int32).astype(jnp.float32)

    fg = jnp.dot(XgI, W['Win'])
    fb = jnp.dot(XbI, W['Win'])
    qg = jnp.dot(XgI, WinWq)
    qb = jnp.dot(XbI, WinWq)
    kg = jnp.dot(XgI, WinWk)
    kb = jnp.dot(XbI, WinWk)
    vg = jnp.dot(XgI, WinWv)
    vb = jnp.dot(XbI, WinWv)

    h2g = jax.nn.relu(jnp.dot(jax.nn.relu(jnp.dot(Xg, W['Wd1'])), W['Wd2']))
    h2b = jax.nn.relu(jnp.dot(jax.nn.relu(jnp.dot(Xb, W['Wd1'])), W['Wd2']))

    io_i = jax.lax.broadcasted_iota(jnp.int32, (N, N), 0)
    io_j = jax.lax.broadcasted_iota(jnp.int32, (N, N), 1)
    ch = jax.lax.broadcasted_iota(jnp.int32, (N, 128), 1)
    icol = io_i[:, 0:1]
    io_jf = io_j.astype(jnp.float32)

    for s in range(SPS):
        sl = slice(s * N_GOOD, (s + 1) * N_GOOD)
        datanum = datanums[s]

        hmax = jnp.maximum(jnp.max(h2g[sl], axis=0, keepdims=True),
                           jnp.max(h2b[sl], axis=0, keepdims=True))
        hmean = (jnp.sum(h2g[sl], axis=0, keepdims=True) +
                 jnp.sum(h2b[sl], axis=0, keepdims=True)) * (1.0 / N)
        pooled = jnp.concatenate([hmax, hmean], axis=1)
        pt_row = jax.nn.sigmoid(jnp.dot(pooled, W['wc']))

        feats = jnp.concatenate([fg[sl], fb[sl]], axis=0) + temb_all[s:s + 1]
        q = jnp.concatenate([qg[sl], qb[sl]], axis=0) + tq[s:s + 1]
        k = jnp.concatenate([kg[sl], kb[sl]], axis=0) + tk[s:s + 1]
        v = jnp.concatenate([vg[sl], vb[sl]], axis=0) + tv[s:s + 1]
        scores = jax.lax.dot_general(q, k, (((1,), (1,)), ((), ())))
        e = jnp.exp(scores)
        den = jnp.sum(e, axis=1, keepdims=True)
        av = jnp.dot(e, v)
        out = feats + jnp.dot(av, W['Wo']) / den
        logits = jnp.dot(jax.nn.relu(out), W['Wh'])

        keys_j = (jnp.broadcast_to(pn_ref[s], (N, N)) +
                  jnp.where(io_j >= datanum, 1e6, 0.0))
        keys_i = keys_j.T
        before = (keys_j < keys_i) | ((keys_j == keys_i) & (io_j < io_i))
        rank = jnp.sum(before.astype(jnp.float32), axis=1, keepdims=True)
        Q = (rank == io_jf).astype(jnp.float32)

        lab = jnp.where(icol < N_GOOD, 0.0, jnp.where(icol < datanum, 1.0, -1.0))
        M = logits + jnp.where(ch[:, 0:8] == 2, lab, 0.0)

        final = jax.lax.dot_general(M, Q, (((0,), (0,)), ((), ())))

        pl_ref[s] = final[0:2, :]
        gt_ref[s] = jnp.round(final[2:3, :]).astype(jnp.int32)
        pt_ref[s] = pt_row[0, 0]
        gtt_ref[s] = ts[s].astype(jnp.float32) * (1.0 / TIMESTEPS)


def kernel(good_tokens, bad_tokens, t, perm_noise,
           W_in, W_t, W_q, W_k, W_v, W_o, W_head, W_d1, W_d2, w_cls):
    tab = _ratio_table()
    pn_row = perm_noise.reshape(B, 1, N)
    Wh128 = jnp.pad(W_head, ((0, 0), (0, 6)))
    wc128 = jnp.pad(w_cls, ((0, 0), (0, 7)))

    full2d = lambda s: pl.BlockSpec(s, lambda i, *_: (0, 0))
    per_b = lambda s: pl.BlockSpec(s, lambda i, *_: (i, 0, 0))

    grid_spec = pltpu.PrefetchScalarGridSpec(
        num_scalar_prefetch=1,
        grid=(B // SPS,),
        in_specs=[
            per_b((SPS, N_GOOD, D)),
            per_b((SPS, N - N_GOOD, D)),
            per_b((SPS, 1, N)),
            full2d((8, 128)),
            full2d((D, DM)),
            full2d((DM, DM)),
            full2d((DM, DM)),
            full2d((DM, DM)),
            full2d((DM, DM)),
            full2d((DM, DM)),
            full2d((DM, 8)),
            full2d((D, 128)),
            full2d((128, 128)),
            full2d((256, 8)),
        ],
        out_specs=[
            per_b((SPS, 2, N)),
            per_b((SPS, 1, N)),
            pl.BlockSpec(memory_space=pltpu.SMEM),
            pl.BlockSpec(memory_space=pltpu.SMEM),
        ],
    )

    pred_label, gt3, pred_t, gt_t = pl.pallas_call(
        _kern,
        grid_spec=grid_spec,
        out_shape=[
            jax.ShapeDtypeStruct((B, 2, N), jnp.float32),
            jax.ShapeDtypeStruct((B, 1, N), jnp.int32),
            jax.ShapeDtypeStruct((B,), jnp.float32),
            jax.ShapeDtypeStruct((B,), jnp.float32),
        ],
    )(t, good_tokens, bad_tokens, pn_row, tab,
      W_in, W_t, W_q, W_k, W_v, W_o, Wh128, W_d1, W_d2, wc128)

    gt_label = gt3.reshape(B, N)
    return pred_label, gt_label, pred_t, gt_t
